# baseline (speedup 1.0000x reference)
import functools

import jax
import jax.numpy as jnp
from jax import lax
from jax.experimental import pallas as pl
from jax.experimental.pallas import tpu as pltpu
from jax.experimental.pallas import tpu_sc as plsc

N_FACTORS = 32
BATCH = 16384
NUM_WORKERS = 32
BPW = BATCH // NUM_WORKERS
LANES = 16
GROUPS = BPW // LANES


def _sc_body(user_hbm, movie_hbm, uf_hbm, mf_hbm, ub_hbm, mb_hbm, out_hbm,
             uidx_v, midx_v, urows_v, mrows_v, ubias_v, mbias_v, out_v, sem):
    c = lax.axis_index("c")
    s = lax.axis_index("s")
    wid = s * 2 + c
    base = wid * BPW

    pltpu.sync_copy(user_hbm.at[pl.ds(base, BPW)], uidx_v)
    pltpu.sync_copy(movie_hbm.at[pl.ds(base, BPW)], midx_v)
    cp1 = pltpu.async_copy(uf_hbm.at[uidx_v], urows_v, sem)
    cp2 = pltpu.async_copy(mf_hbm.at[midx_v], mrows_v, sem)
    cp3 = pltpu.async_copy(ub_hbm.at[uidx_v], ubias_v, sem)
    cp4 = pltpu.async_copy(mb_hbm.at[midx_v], mbias_v, sem)
    cp1.wait()
    cp2.wait()
    cp3.wait()
    cp4.wait()

    lane = lax.iota(jnp.int32, LANES)
    zero = jnp.zeros((LANES,), jnp.int32)

    def group(g, carry):
        row = g * LANES + lane
        acc = jnp.zeros((LANES,), jnp.float32)
        for k in range(N_FACTORS):
            col = jnp.full((LANES,), k, jnp.int32)
            u = plsc.load_gather(urows_v, [row, col])
            m = plsc.load_gather(mrows_v, [row, col])
            acc = acc + u * m
        sl = pl.ds(g * LANES, LANES)
        out_v[sl] = acc + ubias_v[sl] + mbias_v[sl]
        return carry

    lax.fori_loop(0, GROUPS, group, 0)

    pltpu.sync_copy(out_v, out_hbm.at[pl.ds(base, BPW)])


@jax.jit
def _run(user, movie, user_factors, movie_factors, user_biases, movie_biases):
    mesh = plsc.VectorSubcoreMesh(core_axis_name="c", subcore_axis_name="s")
    kern = pl.kernel(
        _sc_body,
        out_type=jax.ShapeDtypeStruct((BATCH,), jnp.float32),
        mesh=mesh,
        scratch_types=[
            pltpu.VMEM((BPW,), jnp.int32),
            pltpu.VMEM((BPW,), jnp.int32),
            pltpu.VMEM((BPW, N_FACTORS), jnp.float32),
            pltpu.VMEM((BPW, N_FACTORS), jnp.float32),
            pltpu.VMEM((BPW,), jnp.float32),
            pltpu.VMEM((BPW,), jnp.float32),
            pltpu.VMEM((BPW,), jnp.float32),
            pltpu.SemaphoreType.DMA,
        ],
        compiler_params=pltpu.CompilerParams(
            needs_layout_passes=False, use_tc_tiling_on_sc=False),
    )
    return kern(user, movie, user_factors, movie_factors,
                user_biases, movie_biases)


def kernel(user, movie, user_factors, movie_factors, user_biases, movie_biases):
    return _run(user.astype(jnp.int32), movie.astype(jnp.int32),
                user_factors, movie_factors,
                user_biases.reshape(-1), movie_biases.reshape(-1))
